# baseline (speedup 1.0000x reference)
import functools

import jax
import jax.numpy as jnp
from jax import lax
from jax.experimental import pallas as pl
from jax.experimental.pallas import tpu as pltpu
from jax.experimental.pallas import tpu_sc as plsc

MAXP = 1000
D = 768
D4 = 192
DW = D // 2
NC, NS = 2, 16
NW = NC * NS
CT = 32
NSEG = 4


def _prep_body(px, py, pw, ph, fw, pjw8, pjb, fb, tall_ref, wc_ref):
    w2 = fw[pl.ds(D, D), :]
    b_tot = jnp.dot(pjb[...], w2, preferred_element_type=jnp.float32) + fb[...]
    tabs = (px, py, pw, ph)
    for k in range(4):
        w1k = fw[pl.ds(k * D4, D4), :]
        tk = jnp.dot(tabs[k][...], w1k, preferred_element_type=jnp.float32)
        if k == 0:
            tk = tk + b_tot
        tall_ref[pl.ds(k * MAXP, MAXP), :] = tk
    wc_ref[...] = jnp.dot(pjw8[...], w2, preferred_element_type=jnp.float32)


def _idx_body(gf_ref, idx_ref):
    x = gf_ref[...][:, :4]
    iv = jnp.clip(x.astype(jnp.int32), 0, MAXP - 1)
    off = lax.broadcasted_iota(jnp.int32, iv.shape, 1) * MAXP
    idx_ref[...] = iv + off


def _cont_body(gf8_ref, wc_ref, g_ref, out_ref):
    x = g_ref[...]
    lo = lax.bitcast_convert_type(x << 16, jnp.float32)
    hi = lax.bitcast_convert_type(x & jnp.int32(-65536), jnp.float32)
    out_ref[...] = jnp.concatenate([lo, hi], axis=1) + jnp.dot(
        gf8_ref[...], wc_ref[...], preferred_element_type=jnp.float32)


def _make_gather_kernel(T):
    TPW = T // NW
    SEG = TPW // NSEG
    NCHUNK = SEG // CT
    assert NCHUNK % 2 == 0

    mesh = plsc.VectorSubcoreMesh(
        core_axis_name="c", subcore_axis_name="s",
        num_cores=NC, num_subcores=NS)

    @functools.partial(
        pl.kernel, mesh=mesh,
        out_type=jax.ShapeDtypeStruct((T, DW), jnp.int32),
        scratch_types=[
            pltpu.VMEM((SEG * 4,), jnp.int32),
            pltpu.VMEM((4 * CT,), jnp.int32),
            pltpu.VMEM((4 * CT,), jnp.int32),
            pltpu.VMEM((2, 4 * CT, DW), jnp.int32),
            pltpu.VMEM((2, CT, DW), jnp.int32),
            pltpu.SemaphoreType.DMA,
            pltpu.SemaphoreType.DMA,
            pltpu.SemaphoreType.DMA,
            pltpu.SemaphoreType.DMA,
        ],
    )
    def gather_sum(tall_hbm, idx_hbm, out_hbm, strip_v, idx0_v, idx1_v,
                   gat_v, acc_v, gsem0, gsem1, osem0, osem1):
        idxs = (idx0_v, idx1_v)
        gsem = (gsem0, gsem1)
        osem = (osem0, osem1)
        wid = lax.axis_index("s") * NC + lax.axis_index("c")
        base0 = wid * TPW

        def fire(ci, slot):
            for j in range(4 * CT // 16):
                idxs[slot][pl.ds(j * 16, 16)] = (
                    strip_v[pl.ds(ci * 4 * CT + j * 16, 16)])
            return pltpu.async_copy(
                tall_hbm.at[idxs[slot]], gat_v.at[slot], gsem[slot])

        def sum_chunk(slot):
            gbf = gat_v.at[slot].bitcast(jnp.bfloat16)
            abf = acc_v.at[slot].bitcast(jnp.bfloat16)

            def tok(t, c2):
                t8 = pl.multiple_of(t * 8, 8)
                t2 = pl.multiple_of(t * 2, 2)
                for r in range(DW // 16):
                    cs = pl.ds(r * 16, 16)
                    a = gbf[pl.ds(t8 + 0, 2), cs]
                    b = gbf[pl.ds(t8 + 2, 2), cs]
                    c = gbf[pl.ds(t8 + 4, 2), cs]
                    d = gbf[pl.ds(t8 + 6, 2), cs]
                    abf[pl.ds(t2, 2), cs] = (a + b) + (c + d)
                return c2
            lax.fori_loop(0, CT, tok, 0)

        def drain_out(slot):
            pltpu.make_async_copy(
                acc_v.at[slot],
                out_hbm.at[pl.ds(base0, CT)],
                osem[slot]).wait()

        for seg in range(NSEG):
            segbase = base0 + seg * SEG
            pltpu.sync_copy(idx_hbm.at[pl.ds(segbase * 4, SEG * 4)], strip_v)
            fire(0, 0).wait()

            def pair(p, carry):
                c0 = p * 2
                for s in range(2):
                    ci = c0 + s
                    nxt = ci + 1

                    @pl.when(nxt < NCHUNK)
                    def _():
                        fire(nxt, (s + 1) % 2)

                    @pl.when(ci > 0)
                    def _():
                        pltpu.make_async_copy(
                            tall_hbm.at[idxs[s]],
                            gat_v.at[s], gsem[s]).wait()

                    if seg == 0:
                        @pl.when(ci >= 2)
                        def _():
                            drain_out(s)
                    else:
                        drain_out(s)

                    sum_chunk(s)

                    pltpu.async_copy(
                        acc_v.at[s],
                        out_hbm.at[pl.ds(segbase + ci * CT, CT)],
                        osem[s])
                return carry

            lax.fori_loop(0, NCHUNK // 2, pair, 0)
        drain_out(0)
        drain_out(1)

    return gather_sum


def kernel(geometric_features, pos_x_table, pos_y_table, width_table,
           height_table, proj_w, proj_b, fusion_w, fusion_b):
    B, N, F = geometric_features.shape
    T = B * N
    gff = geometric_features.reshape(T, F)
    gf8 = jnp.concatenate(
        [gff, jnp.zeros((T, 8 - F), dtype=gff.dtype)], axis=1)
    pjw8 = jnp.concatenate(
        [proj_w, jnp.zeros((8 - F, D), dtype=proj_w.dtype)], axis=0)
    pjb2 = proj_b.reshape(1, D)
    fb2 = fusion_b.reshape(1, D)

    tall, wc = pl.pallas_call(
        _prep_body,
        out_shape=[jax.ShapeDtypeStruct((4 * MAXP, D), jnp.float32),
                   jax.ShapeDtypeStruct((8, D), jnp.float32)],
    )(pos_x_table, pos_y_table, width_table, height_table,
      fusion_w, pjw8, pjb2, fb2)

    tall_bf = tall.astype(jnp.bfloat16)
    tall_packed = jax.lax.bitcast_convert_type(
        jnp.stack([tall_bf[:, :DW], tall_bf[:, DW:]], axis=-1),
        jnp.int32)

    BI = 4096
    idx_all = pl.pallas_call(
        _idx_body,
        grid=(T // BI,),
        in_specs=[pl.BlockSpec((BI, F), lambda i: (i, 0))],
        out_specs=pl.BlockSpec((BI, 4), lambda i: (i, 0)),
        out_shape=jax.ShapeDtypeStruct((T, 4), jnp.int32),
    )(gff)

    g = _make_gather_kernel(T)(tall_packed, idx_all.reshape(T * 4))

    BM = 2048
    out = pl.pallas_call(
        _cont_body,
        grid=(T // BM,),
        in_specs=[pl.BlockSpec((BM, 8), lambda i: (i, 0)),
                  pl.BlockSpec((8, D), lambda i: (0, 0)),
                  pl.BlockSpec((BM, DW), lambda i: (i, 0))],
        out_specs=pl.BlockSpec((BM, D), lambda i: (i, 0)),
        out_shape=jax.ShapeDtypeStruct((T, D), jnp.float32),
    )(gf8, wc, g)

    return out.reshape(B, N, D)


# speedup vs baseline: 1.0594x; 1.0594x over previous
import functools

import jax
import jax.numpy as jnp
from jax import lax
from jax.experimental import pallas as pl
from jax.experimental.pallas import tpu as pltpu
from jax.experimental.pallas import tpu_sc as plsc

MAXP = 1000
D = 768
D4 = 192
DW = D // 2
NC, NS = 2, 16
NW = NC * NS
CT = 32
NSEG = 4


def _prep_body(px, py, pw, ph, fw, pjw8, pjb, fb, tall_ref, wc_ref):
    w2 = fw[pl.ds(D, D), :]
    b_tot = jnp.dot(pjb[...], w2, preferred_element_type=jnp.float32) + fb[...]
    tabs = (px, py, pw, ph)
    for k in range(4):
        w1k = fw[pl.ds(k * D4, D4), :]
        tk = jnp.dot(tabs[k][...], w1k, preferred_element_type=jnp.float32)
        if k == 0:
            tk = tk + b_tot
        tall_ref[pl.ds(k * MAXP, MAXP), :] = tk
    wc_ref[...] = jnp.dot(pjw8[...], w2, preferred_element_type=jnp.float32)


def _idx_body(gf_ref, idx_ref):
    x = gf_ref[...][:, :4]
    iv = jnp.clip(x.astype(jnp.int32), 0, MAXP - 1)
    off = lax.broadcasted_iota(jnp.int32, iv.shape, 1) * MAXP
    idx_ref[...] = iv + off


def _cont_body(gf8_ref, wc_ref, g_ref, out_ref):
    x = g_ref[...]
    lo = lax.bitcast_convert_type(x << 16, jnp.float32)
    hi = lax.bitcast_convert_type(x & jnp.int32(-65536), jnp.float32)
    out_ref[...] = jnp.concatenate([lo, hi], axis=1) + jnp.dot(
        gf8_ref[...], wc_ref[...], preferred_element_type=jnp.float32)


def _cont_body2(gf8_ref, wc_ref, g_ref, prev_ref, out_ref):
    del prev_ref
    _cont_body(gf8_ref, wc_ref, g_ref, out_ref)


def _make_gather_kernel(T):
    TPW = T // NW
    SEG = 1600
    NSEG_L = TPW // SEG
    NCHUNK = SEG // CT
    assert TPW % SEG == 0 and NCHUNK % 2 == 0

    mesh = plsc.VectorSubcoreMesh(
        core_axis_name="c", subcore_axis_name="s",
        num_cores=NC, num_subcores=NS)

    @functools.partial(
        pl.kernel, mesh=mesh,
        out_type=jax.ShapeDtypeStruct((T, DW), jnp.int32),
        scratch_types=[
            pltpu.VMEM((SEG * 4,), jnp.int32),
            pltpu.VMEM((4 * CT,), jnp.int32),
            pltpu.VMEM((4 * CT,), jnp.int32),
            pltpu.VMEM((2, 4 * CT, DW), jnp.int32),
            pltpu.VMEM((2, CT, DW), jnp.int32),
            pltpu.SemaphoreType.DMA,
            pltpu.SemaphoreType.DMA,
            pltpu.SemaphoreType.DMA,
            pltpu.SemaphoreType.DMA,
        ],
    )
    def gather_sum(tall_hbm, idx_hbm, out_hbm, strip_v, idx0_v, idx1_v,
                   gat_v, acc_v, gsem0, gsem1, osem0, osem1):
        idxs = (idx0_v, idx1_v)
        gsem = (gsem0, gsem1)
        osem = (osem0, osem1)
        wid = lax.axis_index("s") * NC + lax.axis_index("c")
        base0 = wid * TPW

        def fire(ci, slot):
            for j in range(4 * CT // 16):
                idxs[slot][pl.ds(j * 16, 16)] = (
                    strip_v[pl.ds(ci * 4 * CT + j * 16, 16)])
            return pltpu.async_copy(
                tall_hbm.at[idxs[slot]], gat_v.at[slot], gsem[slot])

        def sum_chunk(slot):
            gbf = gat_v.at[slot].bitcast(jnp.bfloat16)
            abf = acc_v.at[slot].bitcast(jnp.bfloat16)

            def tok(t, c2):
                t8 = pl.multiple_of(t * 8, 8)
                t2 = pl.multiple_of(t * 2, 2)
                for r in range(DW // 16):
                    cs = pl.ds(r * 16, 16)
                    a = gbf[pl.ds(t8 + 0, 2), cs]
                    b = gbf[pl.ds(t8 + 2, 2), cs]
                    c = gbf[pl.ds(t8 + 4, 2), cs]
                    d = gbf[pl.ds(t8 + 6, 2), cs]
                    abf[pl.ds(t2, 2), cs] = (a + b) + (c + d)
                return c2
            lax.fori_loop(0, CT, tok, 0)

        def drain_out(slot):
            pltpu.make_async_copy(
                acc_v.at[slot],
                out_hbm.at[pl.ds(base0, CT)],
                osem[slot]).wait()

        for seg in range(NSEG_L):
            segbase = base0 + seg * SEG
            pltpu.sync_copy(idx_hbm.at[pl.ds(segbase * 4, SEG * 4)], strip_v)
            fire(0, 0).wait()

            def pair(p, carry):
                c0 = p * 2
                for s in range(2):
                    ci = c0 + s
                    nxt = ci + 1

                    @pl.when(nxt < NCHUNK)
                    def _():
                        fire(nxt, (s + 1) % 2)

                    @pl.when(ci > 0)
                    def _():
                        pltpu.make_async_copy(
                            tall_hbm.at[idxs[s]],
                            gat_v.at[s], gsem[s]).wait()

                    if seg == 0:
                        @pl.when(ci >= 2)
                        def _():
                            drain_out(s)
                    else:
                        drain_out(s)

                    sum_chunk(s)

                    pltpu.async_copy(
                        acc_v.at[s],
                        out_hbm.at[pl.ds(segbase + ci * CT, CT)],
                        osem[s])
                return carry

            lax.fori_loop(0, NCHUNK // 2, pair, 0)
        drain_out(0)
        drain_out(1)

    return gather_sum


def kernel(geometric_features, pos_x_table, pos_y_table, width_table,
           height_table, proj_w, proj_b, fusion_w, fusion_b):
    B, N, F = geometric_features.shape
    T = B * N
    gff = geometric_features.reshape(T, F)
    gf8 = jnp.concatenate(
        [gff, jnp.zeros((T, 8 - F), dtype=gff.dtype)], axis=1)
    pjw8 = jnp.concatenate(
        [proj_w, jnp.zeros((8 - F, D), dtype=proj_w.dtype)], axis=0)
    pjb2 = proj_b.reshape(1, D)
    fb2 = fusion_b.reshape(1, D)

    tall, wc = pl.pallas_call(
        _prep_body,
        out_shape=[jax.ShapeDtypeStruct((4 * MAXP, D), jnp.float32),
                   jax.ShapeDtypeStruct((8, D), jnp.float32)],
    )(pos_x_table, pos_y_table, width_table, height_table,
      fusion_w, pjw8, pjb2, fb2)

    tall_bf = tall.astype(jnp.bfloat16)
    tall_packed = jax.lax.bitcast_convert_type(
        jnp.stack([tall_bf[:, :DW], tall_bf[:, DW:]], axis=-1),
        jnp.int32)

    BI = 4096
    idx_all = pl.pallas_call(
        _idx_body,
        grid=(T // BI,),
        in_specs=[pl.BlockSpec((BI, F), lambda i: (i, 0))],
        out_specs=pl.BlockSpec((BI, 4), lambda i: (i, 0)),
        out_shape=jax.ShapeDtypeStruct((T, 4), jnp.int32),
    )(gff)

    T2 = T // 2
    idx_flat = idx_all.reshape(T * 4)
    gather_half = _make_gather_kernel(T2)
    g1 = gather_half(tall_packed, idx_flat[:T2 * 4])
    g2 = gather_half(tall_packed, idx_flat[T2 * 4:])

    BM = 2048
    NB = T2 // BM
    out1 = pl.pallas_call(
        _cont_body,
        grid=(NB,),
        in_specs=[pl.BlockSpec((BM, 8), lambda i: (i, 0)),
                  pl.BlockSpec((8, D), lambda i: (0, 0)),
                  pl.BlockSpec((BM, DW), lambda i: (i, 0))],
        out_specs=pl.BlockSpec((BM, D), lambda i: (i, 0)),
        out_shape=jax.ShapeDtypeStruct((T, D), jnp.float32),
    )(gf8, wc, g1)

    out = pl.pallas_call(
        _cont_body2,
        grid=(NB,),
        in_specs=[pl.BlockSpec((BM, 8), lambda i: (i + NB, 0)),
                  pl.BlockSpec((8, D), lambda i: (0, 0)),
                  pl.BlockSpec((BM, DW), lambda i: (i, 0)),
                  pl.BlockSpec((8, D), lambda i: (0, 0))],
        out_specs=pl.BlockSpec((BM, D), lambda i: (i + NB, 0)),
        out_shape=jax.ShapeDtypeStruct((T, D), jnp.float32),
        input_output_aliases={3: 0},
    )(gf8, wc, g2, out1)

    return out.reshape(B, N, D)


# speedup vs baseline: 1.0615x; 1.0020x over previous
import functools

import jax
import jax.numpy as jnp
from jax import lax
from jax.experimental import pallas as pl
from jax.experimental.pallas import tpu as pltpu
from jax.experimental.pallas import tpu_sc as plsc

MAXP = 1000
D = 768
D4 = 192
DW = D // 2
NC, NS = 2, 16
NW = NC * NS
CT = 32
NSEG = 4


def _prep_body(px, py, pw, ph, fw, pjw8, pjb, fb, tall_ref, wc_ref):
    w2 = fw[pl.ds(D, D), :]
    b_tot = jnp.dot(pjb[...], w2, preferred_element_type=jnp.float32) + fb[...]
    tabs = (px, py, pw, ph)
    for k in range(4):
        w1k = fw[pl.ds(k * D4, D4), :]
        tk = jnp.dot(tabs[k][...], w1k, preferred_element_type=jnp.float32)
        if k == 0:
            tk = tk + b_tot
        tall_ref[pl.ds(k * MAXP, MAXP), :] = tk
    wc_ref[...] = jnp.dot(pjw8[...], w2, preferred_element_type=jnp.float32)


def _idx_body(gf_ref, idx_ref):
    x = gf_ref[...][:, :4]
    iv = jnp.clip(x.astype(jnp.int32), 0, MAXP - 1)
    off = lax.broadcasted_iota(jnp.int32, iv.shape, 1) * MAXP
    idx_ref[...] = iv + off


def _cont_body(gf8_ref, wc_ref, g_ref, out_ref):
    x = g_ref[...]
    lo = lax.bitcast_convert_type(x << 16, jnp.float32)
    hi = lax.bitcast_convert_type(x & jnp.int32(-65536), jnp.float32)
    out_ref[...] = jnp.concatenate([lo, hi], axis=1) + jnp.dot(
        gf8_ref[...], wc_ref[...], preferred_element_type=jnp.float32)


def _cont_body2(gf8_ref, wc_ref, g_ref, prev_ref, out_ref):
    del prev_ref
    _cont_body(gf8_ref, wc_ref, g_ref, out_ref)


def _make_gather_kernel(T):
    TPW = T // NW
    SEG = 1600
    NSEG_L = TPW // SEG
    NCHUNK = SEG // CT
    assert TPW % SEG == 0 and NCHUNK % 2 == 0

    mesh = plsc.VectorSubcoreMesh(
        core_axis_name="c", subcore_axis_name="s",
        num_cores=NC, num_subcores=NS)

    @functools.partial(
        pl.kernel, mesh=mesh,
        out_type=jax.ShapeDtypeStruct((T, DW), jnp.int32),
        scratch_types=[
            pltpu.VMEM((SEG * 4,), jnp.int32),
            pltpu.VMEM((4 * CT,), jnp.int32),
            pltpu.VMEM((4 * CT,), jnp.int32),
            pltpu.VMEM((2, 4 * CT, DW), jnp.int32),
            pltpu.VMEM((2, CT, DW), jnp.int32),
            pltpu.SemaphoreType.DMA,
            pltpu.SemaphoreType.DMA,
            pltpu.SemaphoreType.DMA,
            pltpu.SemaphoreType.DMA,
        ],
    )
    def gather_sum(tall_hbm, idx_hbm, out_hbm, strip_v, idx0_v, idx1_v,
                   gat_v, acc_v, gsem0, gsem1, osem0, osem1):
        idxs = (idx0_v, idx1_v)
        gsem = (gsem0, gsem1)
        osem = (osem0, osem1)
        wid = lax.axis_index("s") * NC + lax.axis_index("c")
        base0 = wid * TPW

        def fire(ci, slot):
            for j in range(4 * CT // 16):
                idxs[slot][pl.ds(j * 16, 16)] = (
                    strip_v[pl.ds(ci * 4 * CT + j * 16, 16)])
            return pltpu.async_copy(
                tall_hbm.at[idxs[slot]], gat_v.at[slot], gsem[slot])

        def sum_chunk(slot):
            gbf = gat_v.at[slot].bitcast(jnp.bfloat16)
            abf = acc_v.at[slot].bitcast(jnp.bfloat16)

            def tok(u, c2):
                for h in range(2):
                    t8 = pl.multiple_of(u * 16 + 8 * h, 8)
                    t2 = pl.multiple_of(u * 4 + 2 * h, 2)
                    for r in range(DW // 16):
                        cs = pl.ds(r * 16, 16)
                        a = gbf[pl.ds(t8 + 0, 2), cs]
                        b = gbf[pl.ds(t8 + 2, 2), cs]
                        c = gbf[pl.ds(t8 + 4, 2), cs]
                        d = gbf[pl.ds(t8 + 6, 2), cs]
                        abf[pl.ds(t2, 2), cs] = (a + b) + (c + d)
                return c2
            lax.fori_loop(0, CT // 2, tok, 0)

        def drain_out(slot):
            pltpu.make_async_copy(
                acc_v.at[slot],
                out_hbm.at[pl.ds(base0, CT)],
                osem[slot]).wait()

        for seg in range(NSEG_L):
            segbase = base0 + seg * SEG
            pltpu.sync_copy(idx_hbm.at[pl.ds(segbase * 4, SEG * 4)], strip_v)
            fire(0, 0).wait()

            def pair(p, carry):
                c0 = p * 2
                for s in range(2):
                    ci = c0 + s
                    nxt = ci + 1

                    @pl.when(nxt < NCHUNK)
                    def _():
                        fire(nxt, (s + 1) % 2)

                    @pl.when(ci > 0)
                    def _():
                        pltpu.make_async_copy(
                            tall_hbm.at[pl.ds(0, 4 * CT)],
                            gat_v.at[s], gsem[s]).wait()

                    if seg == 0:
                        @pl.when(ci >= 2)
                        def _():
                            drain_out(s)
                    else:
                        drain_out(s)

                    sum_chunk(s)

                    pltpu.async_copy(
                        acc_v.at[s],
                        out_hbm.at[pl.ds(segbase + ci * CT, CT)],
                        osem[s])
                return carry

            lax.fori_loop(0, NCHUNK // 2, pair, 0)
        drain_out(0)
        drain_out(1)

    return gather_sum


def kernel(geometric_features, pos_x_table, pos_y_table, width_table,
           height_table, proj_w, proj_b, fusion_w, fusion_b):
    B, N, F = geometric_features.shape
    T = B * N
    gff = geometric_features.reshape(T, F)
    gf8 = jnp.concatenate(
        [gff, jnp.zeros((T, 8 - F), dtype=gff.dtype)], axis=1)
    pjw8 = jnp.concatenate(
        [proj_w, jnp.zeros((8 - F, D), dtype=proj_w.dtype)], axis=0)
    pjb2 = proj_b.reshape(1, D)
    fb2 = fusion_b.reshape(1, D)

    tall, wc = pl.pallas_call(
        _prep_body,
        out_shape=[jax.ShapeDtypeStruct((4 * MAXP, D), jnp.float32),
                   jax.ShapeDtypeStruct((8, D), jnp.float32)],
    )(pos_x_table, pos_y_table, width_table, height_table,
      fusion_w, pjw8, pjb2, fb2)

    tall_bf = tall.astype(jnp.bfloat16)
    tall_packed = jax.lax.bitcast_convert_type(
        jnp.stack([tall_bf[:, :DW], tall_bf[:, DW:]], axis=-1),
        jnp.int32)

    BI = 4096
    idx_all = pl.pallas_call(
        _idx_body,
        grid=(T // BI,),
        in_specs=[pl.BlockSpec((BI, F), lambda i: (i, 0))],
        out_specs=pl.BlockSpec((BI, 4), lambda i: (i, 0)),
        out_shape=jax.ShapeDtypeStruct((T, 4), jnp.int32),
    )(gff)

    T2 = T // 2
    idx_flat = idx_all.reshape(T * 4)
    gather_half = _make_gather_kernel(T2)
    g1 = gather_half(tall_packed, idx_flat[:T2 * 4])
    g2 = gather_half(tall_packed, idx_flat[T2 * 4:])

    BM = 2048
    NB = T2 // BM
    out1 = pl.pallas_call(
        _cont_body,
        grid=(NB,),
        in_specs=[pl.BlockSpec((BM, 8), lambda i: (i, 0)),
                  pl.BlockSpec((8, D), lambda i: (0, 0)),
                  pl.BlockSpec((BM, DW), lambda i: (i, 0))],
        out_specs=pl.BlockSpec((BM, D), lambda i: (i, 0)),
        out_shape=jax.ShapeDtypeStruct((T, D), jnp.float32),
    )(gf8, wc, g1)

    out = pl.pallas_call(
        _cont_body2,
        grid=(NB,),
        in_specs=[pl.BlockSpec((BM, 8), lambda i: (i + NB, 0)),
                  pl.BlockSpec((8, D), lambda i: (0, 0)),
                  pl.BlockSpec((BM, DW), lambda i: (i, 0)),
                  pl.BlockSpec((8, D), lambda i: (0, 0))],
        out_specs=pl.BlockSpec((BM, D), lambda i: (i + NB, 0)),
        out_shape=jax.ShapeDtypeStruct((T, D), jnp.float32),
        input_output_aliases={3: 0},
    )(gf8, wc, g2, out1)

    return out.reshape(B, N, D)
